# Initial kernel scaffold; baseline (speedup 1.0000x reference)
import jax
import jax.numpy as jnp
from jax.experimental import pallas as pl


def kernel(pos_u, pos_v, neg_v, U, V):
    raise NotImplementedError("write your pallas kernel here")


# speedup vs baseline: 1.5758x; 1.5758x over previous
import functools

import jax
import jax.numpy as jnp
from jax import lax
from jax.experimental import pallas as pl
from jax.experimental.pallas import tpu as pltpu
from jax.experimental.pallas import tpu_sc as plsc

_B = 16384
_D = 64
_NNEG = 5
_NC = 2
_NS = 16
_NW = _NC * _NS
_BPW = _B // _NW
_CH = 128
_NCHUNK = _BPW // _CH


def _sc_scores(pos_u, pos_v, neg_t, U, V):
    mesh = plsc.VectorSubcoreMesh(core_axis_name="c", subcore_axis_name="s")

    @functools.partial(
        pl.kernel,
        mesh=mesh,
        compiler_params=pltpu.CompilerParams(
            needs_layout_passes=False, use_tc_tiling_on_sc=False),
        out_type=jax.ShapeDtypeStruct((8, _B), jnp.float32),
        scratch_types=[
            pltpu.VMEM((_CH,), jnp.int32),
            pltpu.VMEM((_CH,), jnp.int32),
        ] + [pltpu.VMEM((_CH,), jnp.int32) for _ in range(_NNEG)] + [
            pltpu.VMEM((_CH, _D), jnp.float32),
            pltpu.VMEM((_CH, _D), jnp.float32),
            pltpu.VMEM((_NNEG, _CH, _D), jnp.float32),
            pltpu.VMEM((8, _CH), jnp.float32),
            pltpu.SemaphoreType.DMA,
        ],
    )
    def k(pos_u_hbm, pos_v_hbm, neg_hbm, u_hbm, v_hbm, out_hbm,
          idx_u, idx_v, idx_n0, idx_n1, idx_n2, idx_n3, idx_n4,
          u_rows, v_rows, n_rows, scores, sem):
        idx_n = (idx_n0, idx_n1, idx_n2, idx_n3, idx_n4)
        wid = lax.axis_index("s") * _NC + lax.axis_index("c")
        base = wid * _BPW
        lane = lax.iota(jnp.int32, 16)
        for c in range(_NCHUNK):
            gb = base + c * _CH
            pltpu.sync_copy(pos_u_hbm.at[pl.ds(gb, _CH)], idx_u)
            pltpu.sync_copy(pos_v_hbm.at[pl.ds(gb, _CH)], idx_v)
            for j in range(_NNEG):
                pltpu.sync_copy(neg_hbm.at[pl.ds(j * _B + gb, _CH)], idx_n[j])
            cps = [pltpu.async_copy(u_hbm.at[idx_u], u_rows, sem),
                   pltpu.async_copy(v_hbm.at[idx_v], v_rows, sem)]
            for j in range(_NNEG):
                cps.append(pltpu.async_copy(v_hbm.at[idx_n[j]],
                                            n_rows.at[j], sem))
            for cp in cps:
                cp.wait()

            jvec = [jnp.full((16,), j, jnp.int32) for j in range(_NNEG)]
            zero = jnp.zeros((16,), jnp.float32)

            def group(g, carry):
                rvec = g * 16 + lane

                def dstep(d, accs):
                    cvec = jnp.full((16,), d, jnp.int32)
                    uu = plsc.load_gather(u_rows, [rvec, cvec])
                    vv = plsc.load_gather(v_rows, [rvec, cvec])
                    new = [accs[0] + uu * vv]
                    for j in range(_NNEG):
                        nn = plsc.load_gather(n_rows, [jvec[j], rvec, cvec])
                        new.append(accs[1 + j] + uu * nn)
                    return tuple(new)

                accs = lax.fori_loop(0, _D, dstep, (zero,) * 6)
                for t in range(6):
                    scores[t, pl.ds(g * 16, 16)] = accs[t]
                return carry

            lax.fori_loop(0, _CH // 16, group, 0)
            pltpu.sync_copy(scores, out_hbm.at[:, pl.ds(gb, _CH)])

    return k(pos_u, pos_v, neg_t, U, V)


def _tc_loss(s_ref, out_ref):
    s = s_ref[...]
    pos = jnp.clip(s[0, :], -10.0, 10.0)
    pos_loss = jnp.logaddexp(0.0, -pos)
    neg = jnp.clip(s[1:1 + _NNEG, :], -10.0, 10.0)
    neg_loss = jnp.logaddexp(0.0, neg)
    total = jnp.sum(pos_loss) + jnp.sum(neg_loss)
    out_ref[...] = jnp.full((1, 1), total / _B, dtype=jnp.float32)


def kernel(pos_u, pos_v, neg_v, U, V):
    pos_u = pos_u.astype(jnp.int32)
    pos_v = pos_v.astype(jnp.int32)
    neg_t = neg_v.astype(jnp.int32).T.reshape(-1)
    scores = _sc_scores(pos_u, pos_v, neg_t, U, V)
    loss = pl.pallas_call(
        _tc_loss,
        out_shape=jax.ShapeDtypeStruct((1, 1), jnp.float32),
    )(scores)
    return jnp.reshape(loss, ())


# speedup vs baseline: 2.3100x; 1.4660x over previous
import functools

import jax
import jax.numpy as jnp
from jax import lax
from jax.experimental import pallas as pl
from jax.experimental.pallas import tpu as pltpu
from jax.experimental.pallas import tpu_sc as plsc

_B = 16384
_D = 64
_NNEG = 5
_NC = 2
_NS = 16
_NW = _NC * _NS
_BPW = _B // _NW
_CH = 128
_NCHUNK = _BPW // _CH


def _sc_scores(pos_u, pos_v, neg_t, U, V):
    mesh = plsc.VectorSubcoreMesh(core_axis_name="c", subcore_axis_name="s")

    @functools.partial(
        pl.kernel,
        mesh=mesh,
        compiler_params=pltpu.CompilerParams(needs_layout_passes=False),
        out_type=jax.ShapeDtypeStruct((8, _B), jnp.float32),
        scratch_types=[
            pltpu.VMEM((7 * _CH,), jnp.int32),
            pltpu.VMEM((_CH, _D), jnp.float32),
            pltpu.VMEM((_CH, _D), jnp.float32),
            pltpu.VMEM((_NNEG * _CH, _D), jnp.float32),
            pltpu.VMEM((8, _CH), jnp.float32),
            pltpu.SemaphoreType.DMA,
        ],
    )
    def k(pos_u_hbm, pos_v_hbm, neg_hbm, u_hbm, v_hbm, out_hbm,
          idx_stage, u_rows, v_rows, n_rows, scores, sem):
        wid = lax.axis_index("s") * _NC + lax.axis_index("c")
        base = wid * _BPW
        lane = lax.iota(jnp.int32, 16)
        for c in range(_NCHUNK):
            gb = base + c * _CH
            pltpu.sync_copy(pos_u_hbm.at[pl.ds(gb, _CH)],
                            idx_stage.at[pl.ds(0, _CH)])
            pltpu.sync_copy(pos_v_hbm.at[pl.ds(gb, _CH)],
                            idx_stage.at[pl.ds(_CH, _CH)])
            for j in range(_NNEG):
                pltpu.sync_copy(neg_hbm.at[pl.ds(j * _B + gb, _CH)],
                                idx_stage.at[pl.ds((2 + j) * _CH, _CH)])

            def issue(g, carry):
                vu = idx_stage[pl.ds(g * 16, 16)]
                vv = idx_stage[pl.ds(_CH + g * 16, 16)]
                vn = [idx_stage[pl.ds((2 + j) * _CH + g * 16, 16)]
                      for j in range(_NNEG)]
                for l in range(16):
                    b = g * 16 + l
                    pltpu.async_copy(u_hbm.at[pl.ds(vu[l], 1), :],
                                     u_rows.at[pl.ds(b, 1), :], sem)
                    pltpu.async_copy(v_hbm.at[pl.ds(vv[l], 1), :],
                                     v_rows.at[pl.ds(b, 1), :], sem)
                    for j in range(_NNEG):
                        pltpu.async_copy(
                            v_hbm.at[pl.ds(vn[j][l], 1), :],
                            n_rows.at[pl.ds(j * _CH + b, 1), :], sem)
                return carry

            lax.fori_loop(0, _CH // 16, issue, 0)
            pltpu.make_async_copy(u_hbm.at[pl.ds(0, _CH), :], u_rows, sem).wait()
            pltpu.make_async_copy(v_hbm.at[pl.ds(0, _CH), :], v_rows, sem).wait()
            pltpu.make_async_copy(v_hbm.at[pl.ds(0, _NNEG * _CH), :],
                                  n_rows, sem).wait()

            zero = jnp.zeros((16,), jnp.float32)

            def group(g, carry):
                rvec = g * 16 + lane

                def dstep(d, accs):
                    cvec = jnp.full((16,), d, jnp.int32)
                    uu = plsc.load_gather(u_rows, [rvec, cvec])
                    vv = plsc.load_gather(v_rows, [rvec, cvec])
                    new = [accs[0] + uu * vv]
                    for j in range(_NNEG):
                        nn = plsc.load_gather(n_rows, [j * _CH + rvec, cvec])
                        new.append(accs[1 + j] + uu * nn)
                    return tuple(new)

                accs = lax.fori_loop(0, _D, dstep, (zero,) * 6)
                for t in range(6):
                    scores[t, pl.ds(g * 16, 16)] = accs[t]
                return carry

            lax.fori_loop(0, _CH // 16, group, 0)
            pltpu.sync_copy(scores, out_hbm.at[:, pl.ds(gb, _CH)])

    return k(pos_u, pos_v, neg_t, U, V)


def _tc_loss(s_ref, out_ref):
    s = s_ref[...]
    pos = jnp.clip(s[0, :], -10.0, 10.0)
    pos_loss = jnp.logaddexp(0.0, -pos)
    neg = jnp.clip(s[1:1 + _NNEG, :], -10.0, 10.0)
    neg_loss = jnp.logaddexp(0.0, neg)
    total = jnp.sum(pos_loss) + jnp.sum(neg_loss)
    out_ref[...] = jnp.full((1, 1), total / _B, dtype=jnp.float32)


def kernel(pos_u, pos_v, neg_v, U, V):
    pos_u = pos_u.astype(jnp.int32)
    pos_v = pos_v.astype(jnp.int32)
    neg_t = neg_v.astype(jnp.int32).T.reshape(-1)
    scores = _sc_scores(pos_u, pos_v, neg_t, U, V)
    loss = pl.pallas_call(
        _tc_loss,
        out_shape=jax.ShapeDtypeStruct((1, 1), jnp.float32),
    )(scores)
    return jnp.reshape(loss, ())
